# Initial kernel scaffold; baseline (speedup 1.0000x reference)
import functools
import jax
import jax.numpy as jnp
from jax import lax
from jax.experimental import pallas as pl
from jax.experimental.pallas import tpu as pltpu
from jax.experimental.pallas import tpu_sc as plsc

N = 50000
E = 800000
D = 64

BLK = 2000

CH = 128
NSUB = 16
NCHUNK = 392
EPAD = CH * NSUB * NCHUNK
HALF = N // 2
SH_ROWS = 26624
DUMP = HALF
ZCH = 1664
WB = 200


def _mm_body(x_ref, w_ref, o_ref):
    o_ref[...] = jnp.dot(x_ref[...], w_ref[...],
                         preferred_element_type=jnp.float32)


def _matmul(x, w):
    return pl.pallas_call(
        _mm_body,
        grid=(N // BLK,),
        in_specs=[pl.BlockSpec((BLK, D), lambda i: (i, 0)),
                  pl.BlockSpec((D, D), lambda i: (0, 0))],
        out_specs=pl.BlockSpec((BLK, D), lambda i: (i, 0)),
        out_shape=jax.ShapeDtypeStruct((N, D), jnp.float32),
    )(x, w)


def _gru_mm_body(agg_ref, h_ref, wihT_ref, whhT_ref, bih_ref, bhh_ref,
                 w2_ref, x_ref, m_ref):
    agg = agg_ref[...]
    h = h_ref[...]
    gi = jnp.dot(agg, wihT_ref[...],
                 preferred_element_type=jnp.float32) + bih_ref[...]
    gh = jnp.dot(h, whhT_ref[...],
                 preferred_element_type=jnp.float32) + bhh_ref[...]
    r = jax.nn.sigmoid(gi[:, :D] + gh[:, :D])
    z = jax.nn.sigmoid(gi[:, D:2 * D] + gh[:, D:2 * D])
    n = jnp.tanh(gi[:, 2 * D:] + r * gh[:, 2 * D:])
    x_new = (1.0 - z) * n + z * h
    x_ref[...] = x_new
    m_ref[...] = jnp.dot(x_new, w2_ref[...],
                         preferred_element_type=jnp.float32)


def _gru_mm(agg, h, wihT, whhT, bih, bhh, w2):
    return pl.pallas_call(
        _gru_mm_body,
        grid=(N // BLK,),
        in_specs=[pl.BlockSpec((BLK, D), lambda i: (i, 0)),
                  pl.BlockSpec((BLK, D), lambda i: (i, 0)),
                  pl.BlockSpec((D, 3 * D), lambda i: (0, 0)),
                  pl.BlockSpec((D, 3 * D), lambda i: (0, 0)),
                  pl.BlockSpec((1, 3 * D), lambda i: (0, 0)),
                  pl.BlockSpec((1, 3 * D), lambda i: (0, 0)),
                  pl.BlockSpec((D, D), lambda i: (0, 0))],
        out_specs=[pl.BlockSpec((BLK, D), lambda i: (i, 0)),
                   pl.BlockSpec((BLK, D), lambda i: (i, 0))],
        out_shape=[jax.ShapeDtypeStruct((N, D), jnp.float32),
                   jax.ShapeDtypeStruct((N, D), jnp.float32)],
    )(agg, h, wihT, whhT, bih, bhh, w2)


def _gru_sum_body(agg_ref, h_ref, wihT_ref, whhT_ref, bih_ref, bhh_ref,
                  o_ref):
    agg = agg_ref[...]
    h = h_ref[...]
    gi = jnp.dot(agg, wihT_ref[...],
                 preferred_element_type=jnp.float32) + bih_ref[...]
    gh = jnp.dot(h, whhT_ref[...],
                 preferred_element_type=jnp.float32) + bhh_ref[...]
    r = jax.nn.sigmoid(gi[:, :D] + gh[:, :D])
    z = jax.nn.sigmoid(gi[:, D:2 * D] + gh[:, D:2 * D])
    n = jnp.tanh(gi[:, 2 * D:] + r * gh[:, 2 * D:])
    x_new = (1.0 - z) * n + z * h

    @pl.when(pl.program_id(0) == 0)
    def _init():
        o_ref[...] = jnp.zeros_like(o_ref)

    o_ref[...] += jnp.sum(x_new, axis=0, keepdims=True)


def _gru_sum(agg, h, wihT, whhT, bih, bhh):
    return pl.pallas_call(
        _gru_sum_body,
        grid=(N // BLK,),
        in_specs=[pl.BlockSpec((BLK, D), lambda i: (i, 0)),
                  pl.BlockSpec((BLK, D), lambda i: (i, 0)),
                  pl.BlockSpec((D, 3 * D), lambda i: (0, 0)),
                  pl.BlockSpec((D, 3 * D), lambda i: (0, 0)),
                  pl.BlockSpec((1, 3 * D), lambda i: (0, 0)),
                  pl.BlockSpec((1, 3 * D), lambda i: (0, 0))],
        out_specs=pl.BlockSpec((1, D), lambda i: (0, 0)),
        out_shape=jax.ShapeDtypeStruct((1, D), jnp.float32),
    )(agg, h, wihT, whhT, bih, bhh)


@functools.partial(
    pl.kernel,
    mesh=plsc.VectorSubcoreMesh(core_axis_name="c", subcore_axis_name="s"),
    out_type=jax.ShapeDtypeStruct((N, D), jnp.float32),
    scratch_types=[
        pltpu.VMEM((CH,), jnp.int32),
        pltpu.VMEM((CH,), jnp.int32),
        pltpu.VMEM((CH,), jnp.int32),
        pltpu.VMEM((CH, D), jnp.float32),
        pltpu.VMEM_SHARED((SH_ROWS, D), jnp.float32),
        pltpu.SemaphoreType.DMA,
    ],
)
def _sc_scatter(m_hbm, src_hbm, dst_hbm, z_hbm, agg_hbm,
                src_v, dst_v, adj_v, rows_v, shared, sem):
    sc = lax.axis_index("c")
    tid = lax.axis_index("s")
    lo = sc * HALF

    pltpu.sync_copy(z_hbm, rows_v)
    for zi in range(ZCH // CH):
        zbase = tid * ZCH + zi * CH
        pltpu.sync_copy(rows_v, shared.at[pl.ds(zbase, CH)])
    plsc.subcore_barrier()

    def body(i, carry):
        base = pl.multiple_of((tid * NCHUNK + i) * CH, CH)
        pltpu.sync_copy(src_hbm.at[pl.ds(base, CH)], src_v)
        pltpu.sync_copy(dst_hbm.at[pl.ds(base, CH)], dst_v)
        for j in range(CH // 16):
            d = dst_v[pl.ds(j * 16, 16)]
            dl = d - lo
            ok = (dl >= 0) & (dl < HALF)
            adj_v[pl.ds(j * 16, 16)] = jnp.where(ok, dl, DUMP)
        pltpu.async_copy(m_hbm.at[src_v], rows_v, sem).wait()
        pltpu.sync_copy(rows_v, shared.at[adj_v], add=True)
        return carry

    lax.fori_loop(0, NCHUNK, body, 0)
    plsc.subcore_barrier()

    for ci in range(8):
        c = ci * 16 + tid

        @pl.when(c < HALF // WB)
        def _():
            pltpu.sync_copy(shared.at[pl.ds(c * WB, WB)],
                            agg_hbm.at[pl.ds(lo + c * WB, WB)])


@jax.jit
def kernel(h1, edge_index1, weight, w_ih, w_hh, b_ih, b_hh, w_out, b_out):
    src = edge_index1[0]
    dst = edge_index1[1]
    pad = EPAD - E
    src_p = jnp.concatenate([src, jnp.zeros((pad,), jnp.int32)])
    dst_p = jnp.concatenate([dst, jnp.full((pad,), N, jnp.int32)])
    zblk = jnp.zeros((CH, D), jnp.float32)

    wihT = w_ih.T
    whhT = w_hh.T
    bih = b_ih.reshape(1, 3 * D)
    bhh = b_hh.reshape(1, 3 * D)

    m0 = _matmul(h1, weight[0])
    agg0 = _sc_scatter(m0, src_p, dst_p, zblk)
    x1, m1 = _gru_mm(agg0, h1, wihT, whhT, bih, bhh, weight[1])
    agg1 = _sc_scatter(m1, src_p, dst_p, zblk)
    colsum = _gru_sum(agg1, x1, wihT, whhT, bih, bhh)

    a2 = colsum @ w_out.T + N * b_out
    return jax.nn.softmax(a2, axis=-1)


# speedup vs baseline: 3.2089x; 3.2089x over previous
import functools
import jax
import jax.numpy as jnp
from jax import lax
from jax.experimental import pallas as pl
from jax.experimental.pallas import tpu as pltpu
from jax.experimental.pallas import tpu_sc as plsc

N = 50000
E = 800000
D = 64

BLK = 2000

CH = 128
NSUB = 16
NCHUNK = 392
EPAD = CH * NSUB * NCHUNK
HALF = N // 2
SH_ROWS = 26624
DUMP = HALF
ZCH = 1664
WB = 200


def _mm_body(x_ref, w_ref, o_ref):
    o_ref[...] = jnp.dot(x_ref[...], w_ref[...],
                         preferred_element_type=jnp.float32)


def _matmul(x, w):
    return pl.pallas_call(
        _mm_body,
        grid=(N // BLK,),
        in_specs=[pl.BlockSpec((BLK, D), lambda i: (i, 0)),
                  pl.BlockSpec((D, D), lambda i: (0, 0))],
        out_specs=pl.BlockSpec((BLK, D), lambda i: (i, 0)),
        out_shape=jax.ShapeDtypeStruct((N, D), jnp.float32),
    )(x, w)


def _gru_mm_body(agg_ref, h_ref, wihT_ref, whhT_ref, bih_ref, bhh_ref,
                 w2_ref, x_ref, m_ref):
    agg = agg_ref[...]
    h = h_ref[...]
    gi = jnp.dot(agg, wihT_ref[...],
                 preferred_element_type=jnp.float32) + bih_ref[...]
    gh = jnp.dot(h, whhT_ref[...],
                 preferred_element_type=jnp.float32) + bhh_ref[...]
    r = jax.nn.sigmoid(gi[:, :D] + gh[:, :D])
    z = jax.nn.sigmoid(gi[:, D:2 * D] + gh[:, D:2 * D])
    n = jnp.tanh(gi[:, 2 * D:] + r * gh[:, 2 * D:])
    x_new = (1.0 - z) * n + z * h
    x_ref[...] = x_new
    m_ref[...] = jnp.dot(x_new, w2_ref[...],
                         preferred_element_type=jnp.float32)


def _gru_mm(agg, h, wihT, whhT, bih, bhh, w2):
    return pl.pallas_call(
        _gru_mm_body,
        grid=(N // BLK,),
        in_specs=[pl.BlockSpec((BLK, D), lambda i: (i, 0)),
                  pl.BlockSpec((BLK, D), lambda i: (i, 0)),
                  pl.BlockSpec((D, 3 * D), lambda i: (0, 0)),
                  pl.BlockSpec((D, 3 * D), lambda i: (0, 0)),
                  pl.BlockSpec((1, 3 * D), lambda i: (0, 0)),
                  pl.BlockSpec((1, 3 * D), lambda i: (0, 0)),
                  pl.BlockSpec((D, D), lambda i: (0, 0))],
        out_specs=[pl.BlockSpec((BLK, D), lambda i: (i, 0)),
                   pl.BlockSpec((BLK, D), lambda i: (i, 0))],
        out_shape=[jax.ShapeDtypeStruct((N, D), jnp.float32),
                   jax.ShapeDtypeStruct((N, D), jnp.float32)],
    )(agg, h, wihT, whhT, bih, bhh, w2)


def _gru_sum_body(agg_ref, h_ref, wihT_ref, whhT_ref, bih_ref, bhh_ref,
                  o_ref):
    agg = agg_ref[...]
    h = h_ref[...]
    gi = jnp.dot(agg, wihT_ref[...],
                 preferred_element_type=jnp.float32) + bih_ref[...]
    gh = jnp.dot(h, whhT_ref[...],
                 preferred_element_type=jnp.float32) + bhh_ref[...]
    r = jax.nn.sigmoid(gi[:, :D] + gh[:, :D])
    z = jax.nn.sigmoid(gi[:, D:2 * D] + gh[:, D:2 * D])
    n = jnp.tanh(gi[:, 2 * D:] + r * gh[:, 2 * D:])
    x_new = (1.0 - z) * n + z * h

    @pl.when(pl.program_id(0) == 0)
    def _init():
        o_ref[...] = jnp.zeros_like(o_ref)

    o_ref[...] += jnp.sum(x_new, axis=0, keepdims=True)


def _gru_sum(agg, h, wihT, whhT, bih, bhh):
    return pl.pallas_call(
        _gru_sum_body,
        grid=(N // BLK,),
        in_specs=[pl.BlockSpec((BLK, D), lambda i: (i, 0)),
                  pl.BlockSpec((BLK, D), lambda i: (i, 0)),
                  pl.BlockSpec((D, 3 * D), lambda i: (0, 0)),
                  pl.BlockSpec((D, 3 * D), lambda i: (0, 0)),
                  pl.BlockSpec((1, 3 * D), lambda i: (0, 0)),
                  pl.BlockSpec((1, 3 * D), lambda i: (0, 0))],
        out_specs=pl.BlockSpec((1, D), lambda i: (0, 0)),
        out_shape=jax.ShapeDtypeStruct((1, D), jnp.float32),
    )(agg, h, wihT, whhT, bih, bhh)


@functools.partial(
    pl.kernel,
    mesh=plsc.VectorSubcoreMesh(core_axis_name="c", subcore_axis_name="s"),
    out_type=jax.ShapeDtypeStruct((N, D), jnp.float32),
    scratch_types=[
        pltpu.VMEM((CH,), jnp.int32),
        pltpu.VMEM((CH,), jnp.int32),
        pltpu.VMEM((CH,), jnp.int32),
        pltpu.VMEM((CH, D), jnp.float32),
        pltpu.VMEM_SHARED((SH_ROWS, D), jnp.float32),
        pltpu.SemaphoreType.DMA,
    ],
    compiler_params=pltpu.CompilerParams(use_tc_tiling_on_sc=False),
)
def _sc_scatter(m_hbm, src_hbm, dst_hbm, z_hbm, agg_hbm,
                src_v, dst_v, adj_v, rows_v, shared, sem):
    sc = lax.axis_index("c")
    tid = lax.axis_index("s")
    lo = sc * HALF

    pltpu.sync_copy(z_hbm, rows_v)
    for zi in range(ZCH // CH):
        zbase = tid * ZCH + zi * CH
        pltpu.sync_copy(rows_v, shared.at[pl.ds(zbase, CH)])
    plsc.subcore_barrier()

    def body(i, carry):
        base = pl.multiple_of((tid * NCHUNK + i) * CH, CH)
        pltpu.sync_copy(src_hbm.at[pl.ds(base, CH)], src_v)
        pltpu.sync_copy(dst_hbm.at[pl.ds(base, CH)], dst_v)
        for j in range(CH // 16):
            d = dst_v[pl.ds(j * 16, 16)]
            dl = d - lo
            ok = (dl >= 0) & (dl < HALF)
            adj_v[pl.ds(j * 16, 16)] = jnp.where(ok, dl, DUMP)
        pltpu.async_copy(m_hbm.at[src_v], rows_v, sem).wait()
        pltpu.sync_copy(rows_v, shared.at[adj_v], add=True)
        return carry

    lax.fori_loop(0, NCHUNK, body, 0)
    plsc.subcore_barrier()

    for ci in range(8):
        c = ci * 16 + tid

        @pl.when(c < HALF // WB)
        def _():
            pltpu.sync_copy(shared.at[pl.ds(c * WB, WB)],
                            agg_hbm.at[pl.ds(lo + c * WB, WB)])


@jax.jit
def kernel(h1, edge_index1, weight, w_ih, w_hh, b_ih, b_hh, w_out, b_out):
    src = edge_index1[0]
    dst = edge_index1[1]
    pad = EPAD - E
    src_p = jnp.concatenate([src, jnp.zeros((pad,), jnp.int32)])
    dst_p = jnp.concatenate([dst, jnp.full((pad,), N, jnp.int32)])
    zblk = jnp.zeros((CH, D), jnp.float32)

    wihT = w_ih.T
    whhT = w_hh.T
    bih = b_ih.reshape(1, 3 * D)
    bhh = b_hh.reshape(1, 3 * D)

    m0 = _matmul(h1, weight[0])
    agg0 = _sc_scatter(m0, src_p, dst_p, zblk)
    x1, m1 = _gru_mm(agg0, h1, wihT, whhT, bih, bhh, weight[1])
    agg1 = _sc_scatter(m1, src_p, dst_p, zblk)
    colsum = _gru_sum(agg1, x1, wihT, whhT, bih, bhh)

    a2 = colsum @ w_out.T + N * b_out
    return jax.nn.softmax(a2, axis=-1)


# speedup vs baseline: 4.6029x; 1.4344x over previous
import functools
import jax
import jax.numpy as jnp
from jax import lax
from jax.experimental import pallas as pl
from jax.experimental.pallas import tpu as pltpu
from jax.experimental.pallas import tpu_sc as plsc

N = 50000
E = 800000
D = 64

BLK = 2000

CH = 128
NSUB = 16
G = 8
NGROUP = 49
NCHUNK = G * NGROUP
EPAD = CH * NSUB * NCHUNK
EROWS = EPAD // CH
HALF = N // 2
SH_ROWS = 25088
DUMP = HALF
WB = 200


def _mm_body(x_ref, w_ref, o_ref):
    o_ref[...] = jnp.dot(x_ref[...], w_ref[...],
                         preferred_element_type=jnp.float32)


def _matmul(x, w):
    return pl.pallas_call(
        _mm_body,
        grid=(N // BLK,),
        in_specs=[pl.BlockSpec((BLK, D), lambda i: (i, 0)),
                  pl.BlockSpec((D, D), lambda i: (0, 0))],
        out_specs=pl.BlockSpec((BLK, D), lambda i: (i, 0)),
        out_shape=jax.ShapeDtypeStruct((N, D), jnp.float32),
    )(x, w)


def _gru_mm_body(agg_ref, h_ref, wihT_ref, whhT_ref, bih_ref, bhh_ref,
                 w2_ref, x_ref, m_ref):
    agg = agg_ref[...]
    h = h_ref[...]
    gi = jnp.dot(agg, wihT_ref[...],
                 preferred_element_type=jnp.float32) + bih_ref[...]
    gh = jnp.dot(h, whhT_ref[...],
                 preferred_element_type=jnp.float32) + bhh_ref[...]
    r = jax.nn.sigmoid(gi[:, :D] + gh[:, :D])
    z = jax.nn.sigmoid(gi[:, D:2 * D] + gh[:, D:2 * D])
    n = jnp.tanh(gi[:, 2 * D:] + r * gh[:, 2 * D:])
    x_new = (1.0 - z) * n + z * h
    x_ref[...] = x_new
    m_ref[...] = jnp.dot(x_new, w2_ref[...],
                         preferred_element_type=jnp.float32)


def _gru_mm(agg, h, wihT, whhT, bih, bhh, w2):
    return pl.pallas_call(
        _gru_mm_body,
        grid=(N // BLK,),
        in_specs=[pl.BlockSpec((BLK, D), lambda i: (i, 0)),
                  pl.BlockSpec((BLK, D), lambda i: (i, 0)),
                  pl.BlockSpec((D, 3 * D), lambda i: (0, 0)),
                  pl.BlockSpec((D, 3 * D), lambda i: (0, 0)),
                  pl.BlockSpec((1, 3 * D), lambda i: (0, 0)),
                  pl.BlockSpec((1, 3 * D), lambda i: (0, 0)),
                  pl.BlockSpec((D, D), lambda i: (0, 0))],
        out_specs=[pl.BlockSpec((BLK, D), lambda i: (i, 0)),
                   pl.BlockSpec((BLK, D), lambda i: (i, 0))],
        out_shape=[jax.ShapeDtypeStruct((N, D), jnp.float32),
                   jax.ShapeDtypeStruct((N, D), jnp.float32)],
    )(agg, h, wihT, whhT, bih, bhh, w2)


def _gru_sum_body(agg_ref, h_ref, wihT_ref, whhT_ref, bih_ref, bhh_ref,
                  o_ref):
    agg = agg_ref[...]
    h = h_ref[...]
    gi = jnp.dot(agg, wihT_ref[...],
                 preferred_element_type=jnp.float32) + bih_ref[...]
    gh = jnp.dot(h, whhT_ref[...],
                 preferred_element_type=jnp.float32) + bhh_ref[...]
    r = jax.nn.sigmoid(gi[:, :D] + gh[:, :D])
    z = jax.nn.sigmoid(gi[:, D:2 * D] + gh[:, D:2 * D])
    n = jnp.tanh(gi[:, 2 * D:] + r * gh[:, 2 * D:])
    x_new = (1.0 - z) * n + z * h

    @pl.when(pl.program_id(0) == 0)
    def _init():
        o_ref[...] = jnp.zeros_like(o_ref)

    o_ref[...] += jnp.sum(x_new, axis=0, keepdims=True)


def _gru_sum(agg, h, wihT, whhT, bih, bhh):
    return pl.pallas_call(
        _gru_sum_body,
        grid=(N // BLK,),
        in_specs=[pl.BlockSpec((BLK, D), lambda i: (i, 0)),
                  pl.BlockSpec((BLK, D), lambda i: (i, 0)),
                  pl.BlockSpec((D, 3 * D), lambda i: (0, 0)),
                  pl.BlockSpec((D, 3 * D), lambda i: (0, 0)),
                  pl.BlockSpec((1, 3 * D), lambda i: (0, 0)),
                  pl.BlockSpec((1, 3 * D), lambda i: (0, 0))],
        out_specs=pl.BlockSpec((1, D), lambda i: (0, 0)),
        out_shape=jax.ShapeDtypeStruct((1, D), jnp.float32),
    )(agg, h, wihT, whhT, bih, bhh)


@functools.partial(
    pl.kernel,
    mesh=plsc.VectorSubcoreMesh(core_axis_name="c", subcore_axis_name="s"),
    out_type=jax.ShapeDtypeStruct((N, D), jnp.float32),
    scratch_types=[
        pltpu.VMEM((G, CH), jnp.int32),
        pltpu.VMEM((G, CH), jnp.int32),
        pltpu.VMEM((G, CH), jnp.int32),
        pltpu.VMEM((G, CH), jnp.int32),
        pltpu.VMEM((G, CH), jnp.int32),
        pltpu.VMEM((G, CH), jnp.int32),
        pltpu.VMEM((CH, D), jnp.float32),
        pltpu.VMEM((CH, D), jnp.float32),
        pltpu.VMEM_SHARED((SH_ROWS, D), jnp.float32),
        pltpu.SemaphoreType.DMA,
        pltpu.SemaphoreType.DMA,
    ],
    compiler_params=pltpu.CompilerParams(use_tc_tiling_on_sc=False),
)
def _sc_scatter(m_hbm, src_hbm, dst_hbm, z_hbm, agg_hbm,
                srcA, dstA, adjA, srcB, dstB, adjB, rows0, rows1,
                shared, sem0, sem1):
    sc = lax.axis_index("c")
    tid = lax.axis_index("s")
    lo = sc * HALF
    rows = (rows0, rows1)
    sems = (sem0, sem1)

    pltpu.sync_copy(z_hbm, rows0)
    for ci in range(13):
        zc = ci * 16 + tid

        @pl.when(zc < SH_ROWS // CH)
        def _():
            pltpu.sync_copy(rows0, shared.at[pl.ds(zc * CH, CH)])
    plsc.subcore_barrier()

    def load_idx(g, src_b, dst_b, adj_b):
        base = tid * NGROUP + g
        pltpu.sync_copy(src_hbm.at[pl.ds(base * G, G), :], src_b)
        pltpu.sync_copy(dst_hbm.at[pl.ds(base * G, G), :], dst_b)
        for j in range(G):
            for i in range(CH // 16):
                d = dst_b[j, pl.ds(i * 16, 16)]
                dl = d - lo
                ok = (dl >= 0) & (dl < HALF)
                adj_b[j, pl.ds(i * 16, 16)] = jnp.where(ok, dl, DUMP)

    def fire(src_b, j, p):
        pltpu.async_copy(m_hbm.at[src_b.at[j]], rows[p], sems[p])

    def drain_scatter(src_b, adj_b, j, p):
        pltpu.make_async_copy(m_hbm.at[src_b.at[j]], rows[p],
                              sems[p]).wait()
        pltpu.sync_copy(rows[p], shared.at[adj_b.at[j]], add=True)

    load_idx(0, srcA, dstA, adjA)
    fire(srcA, 0, 0)

    def body(k, carry):
        load_idx(2 * k + 1, srcB, dstB, adjB)
        for j in range(G):
            if j < G - 1:
                fire(srcA, j + 1, (j + 1) % 2)
            else:
                fire(srcB, 0, (j + 1) % 2)
            drain_scatter(srcA, adjA, j, j % 2)
        load_idx(2 * k + 2, srcA, dstA, adjA)
        for j in range(G):
            if j < G - 1:
                fire(srcB, j + 1, (j + 1) % 2)
            else:
                fire(srcA, 0, (j + 1) % 2)
            drain_scatter(srcB, adjB, j, j % 2)
        return carry

    lax.fori_loop(0, NGROUP // 2, body, 0)

    for j in range(G):
        if j < G - 1:
            fire(srcA, j + 1, (j + 1) % 2)
        drain_scatter(srcA, adjA, j, j % 2)
    plsc.subcore_barrier()

    for ci in range(8):
        c = ci * 16 + tid

        @pl.when(c < HALF // WB)
        def _():
            pltpu.sync_copy(shared.at[pl.ds(c * WB, WB)],
                            agg_hbm.at[pl.ds(lo + c * WB, WB)])


@jax.jit
def kernel(h1, edge_index1, weight, w_ih, w_hh, b_ih, b_hh, w_out, b_out):
    src = edge_index1[0]
    dst = edge_index1[1]
    pad = EPAD - E
    src_p = jnp.concatenate([src, jnp.zeros((pad,), jnp.int32)])
    src_p = src_p.reshape(EROWS, CH)
    dst_p = jnp.concatenate([dst, jnp.full((pad,), N, jnp.int32)])
    dst_p = dst_p.reshape(EROWS, CH)
    zblk = jnp.zeros((CH, D), jnp.float32)

    wihT = w_ih.T
    whhT = w_hh.T
    bih = b_ih.reshape(1, 3 * D)
    bhh = b_hh.reshape(1, 3 * D)

    m0 = _matmul(h1, weight[0])
    agg0 = _sc_scatter(m0, src_p, dst_p, zblk)
    x1, m1 = _gru_mm(agg0, h1, wihT, whhT, bih, bhh, weight[1])
    agg1 = _sc_scatter(m1, src_p, dst_p, zblk)
    colsum = _gru_sum(agg1, x1, wihT, whhT, bih, bhh)

    a2 = colsum @ w_out.T + N * b_out
    return jax.nn.softmax(a2, axis=-1)


# speedup vs baseline: 7.5612x; 1.6427x over previous
import functools
import jax
import jax.numpy as jnp
from jax import lax
from jax.experimental import pallas as pl
from jax.experimental.pallas import tpu as pltpu
from jax.experimental.pallas import tpu_sc as plsc

N = 50000
E = 800000
D = 64
DH = D // 2

BLK = 2000

CH = 128
NSUB = 16
G = 8
NGROUP = 49
NCHUNK = G * NGROUP
EPAD = CH * NSUB * NCHUNK
EROWS = EPAD // CH
SH_ROWS = 50176
NDUMP = SH_ROWS - N
WB = 200


def _mm_body(x_ref, w_ref, o_ref):
    r = jnp.dot(x_ref[...], w_ref[...],
                preferred_element_type=jnp.float32)
    o_ref[0] = r[:, :DH]
    o_ref[1] = r[:, DH:]


def _matmul(x, w):
    return pl.pallas_call(
        _mm_body,
        grid=(N // BLK,),
        in_specs=[pl.BlockSpec((BLK, D), lambda i: (i, 0)),
                  pl.BlockSpec((D, D), lambda i: (0, 0))],
        out_specs=pl.BlockSpec((2, BLK, DH), lambda i: (0, i, 0)),
        out_shape=jax.ShapeDtypeStruct((2, N, DH), jnp.float32),
    )(x, w)


def _gru_mm_body(agg_ref, h_ref, wihT_ref, whhT_ref, bih_ref, bhh_ref,
                 w2_ref, x_ref, m_ref):
    agg = jnp.concatenate([agg_ref[0], agg_ref[1]], axis=1)
    h = h_ref[...]
    gi = jnp.dot(agg, wihT_ref[...],
                 preferred_element_type=jnp.float32) + bih_ref[...]
    gh = jnp.dot(h, whhT_ref[...],
                 preferred_element_type=jnp.float32) + bhh_ref[...]
    r = jax.nn.sigmoid(gi[:, :D] + gh[:, :D])
    z = jax.nn.sigmoid(gi[:, D:2 * D] + gh[:, D:2 * D])
    n = jnp.tanh(gi[:, 2 * D:] + r * gh[:, 2 * D:])
    x_new = (1.0 - z) * n + z * h
    x_ref[...] = x_new
    m = jnp.dot(x_new, w2_ref[...], preferred_element_type=jnp.float32)
    m_ref[0] = m[:, :DH]
    m_ref[1] = m[:, DH:]


def _gru_mm(agg, h, wihT, whhT, bih, bhh, w2):
    return pl.pallas_call(
        _gru_mm_body,
        grid=(N // BLK,),
        in_specs=[pl.BlockSpec((2, BLK, DH), lambda i: (0, i, 0)),
                  pl.BlockSpec((BLK, D), lambda i: (i, 0)),
                  pl.BlockSpec((D, 3 * D), lambda i: (0, 0)),
                  pl.BlockSpec((D, 3 * D), lambda i: (0, 0)),
                  pl.BlockSpec((1, 3 * D), lambda i: (0, 0)),
                  pl.BlockSpec((1, 3 * D), lambda i: (0, 0)),
                  pl.BlockSpec((D, D), lambda i: (0, 0))],
        out_specs=[pl.BlockSpec((BLK, D), lambda i: (i, 0)),
                   pl.BlockSpec((2, BLK, DH), lambda i: (0, i, 0))],
        out_shape=[jax.ShapeDtypeStruct((N, D), jnp.float32),
                   jax.ShapeDtypeStruct((2, N, DH), jnp.float32)],
    )(agg, h, wihT, whhT, bih, bhh, w2)


def _gru_sum_body(agg_ref, h_ref, wihT_ref, whhT_ref, bih_ref, bhh_ref,
                  o_ref):
    agg = jnp.concatenate([agg_ref[0], agg_ref[1]], axis=1)
    h = h_ref[...]
    gi = jnp.dot(agg, wihT_ref[...],
                 preferred_element_type=jnp.float32) + bih_ref[...]
    gh = jnp.dot(h, whhT_ref[...],
                 preferred_element_type=jnp.float32) + bhh_ref[...]
    r = jax.nn.sigmoid(gi[:, :D] + gh[:, :D])
    z = jax.nn.sigmoid(gi[:, D:2 * D] + gh[:, D:2 * D])
    n = jnp.tanh(gi[:, 2 * D:] + r * gh[:, 2 * D:])
    x_new = (1.0 - z) * n + z * h

    @pl.when(pl.program_id(0) == 0)
    def _init():
        o_ref[...] = jnp.zeros_like(o_ref)

    o_ref[...] += jnp.sum(x_new, axis=0, keepdims=True)


def _gru_sum(agg, h, wihT, whhT, bih, bhh):
    return pl.pallas_call(
        _gru_sum_body,
        grid=(N // BLK,),
        in_specs=[pl.BlockSpec((2, BLK, DH), lambda i: (0, i, 0)),
                  pl.BlockSpec((BLK, D), lambda i: (i, 0)),
                  pl.BlockSpec((D, 3 * D), lambda i: (0, 0)),
                  pl.BlockSpec((D, 3 * D), lambda i: (0, 0)),
                  pl.BlockSpec((1, 3 * D), lambda i: (0, 0)),
                  pl.BlockSpec((1, 3 * D), lambda i: (0, 0))],
        out_specs=pl.BlockSpec((1, D), lambda i: (0, 0)),
        out_shape=jax.ShapeDtypeStruct((1, D), jnp.float32),
    )(agg, h, wihT, whhT, bih, bhh)


@functools.partial(
    pl.kernel,
    mesh=plsc.VectorSubcoreMesh(core_axis_name="c", subcore_axis_name="s"),
    out_type=jax.ShapeDtypeStruct((2, N, DH), jnp.float32),
    scratch_types=[
        pltpu.VMEM((G, CH), jnp.int32),
        pltpu.VMEM((G, CH), jnp.int32),
        pltpu.VMEM((G, CH), jnp.int32),
        pltpu.VMEM((G, CH), jnp.int32),
        pltpu.VMEM((CH, DH), jnp.float32),
        pltpu.VMEM((CH, DH), jnp.float32),
        pltpu.VMEM_SHARED((SH_ROWS, DH), jnp.float32),
        pltpu.SemaphoreType.DMA,
        pltpu.SemaphoreType.DMA,
    ],
    compiler_params=pltpu.CompilerParams(use_tc_tiling_on_sc=False),
)
def _sc_scatter(m_hbm, src_hbm, dst_hbm, z_hbm, agg_hbm,
                srcA, dstA, srcB, dstB, rows0, rows1,
                shared, sem0, sem1):
    sc = lax.axis_index("c")
    tid = lax.axis_index("s")
    mview = m_hbm.at[sc]
    aggview = agg_hbm.at[sc]
    rows = (rows0, rows1)
    sems = (sem0, sem1)

    pltpu.sync_copy(z_hbm, rows0)
    for ci in range(25):
        zc = ci * 16 + tid

        @pl.when(zc < SH_ROWS // CH)
        def _():
            pltpu.sync_copy(rows0, shared.at[pl.ds(zc * CH, CH)])
    plsc.subcore_barrier()

    def load_idx(g, src_b, dst_b):
        base = tid * NGROUP + g
        pltpu.sync_copy(src_hbm.at[pl.ds(base * G, G), :], src_b)
        pltpu.sync_copy(dst_hbm.at[pl.ds(base * G, G), :], dst_b)

    def fire(src_b, j, p):
        pltpu.async_copy(mview.at[src_b.at[j]], rows[p], sems[p])

    def drain_scatter(src_b, dst_b, j, p):
        pltpu.make_async_copy(mview.at[src_b.at[j]], rows[p],
                              sems[p]).wait()
        pltpu.sync_copy(rows[p], shared.at[dst_b.at[j]], add=True)

    load_idx(0, srcA, dstA)
    fire(srcA, 0, 0)

    def body(k, carry):
        load_idx(2 * k + 1, srcB, dstB)
        for j in range(G):
            if j < G - 1:
                fire(srcA, j + 1, (j + 1) % 2)
            else:
                fire(srcB, 0, (j + 1) % 2)
            drain_scatter(srcA, dstA, j, j % 2)
        load_idx(2 * k + 2, srcA, dstA)
        for j in range(G):
            if j < G - 1:
                fire(srcB, j + 1, (j + 1) % 2)
            else:
                fire(srcA, 0, (j + 1) % 2)
            drain_scatter(srcB, dstB, j, j % 2)
        return carry

    lax.fori_loop(0, NGROUP // 2, body, 0)

    for j in range(G):
        if j < G - 1:
            fire(srcA, j + 1, (j + 1) % 2)
        drain_scatter(srcA, dstA, j, j % 2)
    plsc.subcore_barrier()

    for ci in range(16):
        c = ci * 16 + tid

        @pl.when(c < N // WB)
        def _():
            pltpu.sync_copy(shared.at[pl.ds(c * WB, WB)],
                            aggview.at[pl.ds(c * WB, WB)])


@jax.jit
def kernel(h1, edge_index1, weight, w_ih, w_hh, b_ih, b_hh, w_out, b_out):
    src = edge_index1[0]
    dst = edge_index1[1]
    pad = EPAD - E
    pad_src = (jnp.arange(pad, dtype=jnp.int32) * 641) % N
    pad_dst = N + (jnp.arange(pad, dtype=jnp.int32) % NDUMP)
    src_p = jnp.concatenate([src, pad_src]).reshape(EROWS, CH)
    dst_p = jnp.concatenate([dst, pad_dst]).reshape(EROWS, CH)
    zblk = jnp.zeros((CH, DH), jnp.float32)

    wihT = w_ih.T
    whhT = w_hh.T
    bih = b_ih.reshape(1, 3 * D)
    bhh = b_hh.reshape(1, 3 * D)

    m0 = _matmul(h1, weight[0])
    agg0 = _sc_scatter(m0, src_p, dst_p, zblk)
    x1, m1 = _gru_mm(agg0, h1, wihT, whhT, bih, bhh, weight[1])
    agg1 = _sc_scatter(m1, src_p, dst_p, zblk)
    colsum = _gru_sum(agg1, x1, wihT, whhT, bih, bhh)

    a2 = colsum @ w_out.T + N * b_out
    return jax.nn.softmax(a2, axis=-1)


# speedup vs baseline: 8.6863x; 1.1488x over previous
import functools
import jax
import jax.numpy as jnp
from jax import lax
from jax.experimental import pallas as pl
from jax.experimental.pallas import tpu as pltpu
from jax.experimental.pallas import tpu_sc as plsc

N = 50000
E = 800000
D = 64
DH = D // 2

BLK = 2000

CH = 128
NSUB = 16
G = 8
NGROUP = 49
NCHUNK = G * NGROUP
EPAD = CH * NSUB * NCHUNK
EROWS = EPAD // CH
SH_ROWS = 50176
NDUMP = SH_ROWS - N
WB = 200


def _mm_body(x_ref, w_ref, o_ref):
    r = jnp.dot(x_ref[...], w_ref[...],
                preferred_element_type=jnp.float32)
    o_ref[0] = r[:, :DH]
    o_ref[1] = r[:, DH:]


def _matmul(x, w):
    return pl.pallas_call(
        _mm_body,
        grid=(N // BLK,),
        in_specs=[pl.BlockSpec((BLK, D), lambda i: (i, 0)),
                  pl.BlockSpec((D, D), lambda i: (0, 0))],
        out_specs=pl.BlockSpec((2, BLK, DH), lambda i: (0, i, 0)),
        out_shape=jax.ShapeDtypeStruct((2, N, DH), jnp.float32),
    )(x, w)


def _gru_mm_body(agg_ref, h_ref, wihT_ref, whhT_ref, bih_ref, bhh_ref,
                 w2_ref, x_ref, m_ref):
    agg = jnp.concatenate([agg_ref[0], agg_ref[1]], axis=1)
    h = h_ref[...]
    gi = jnp.dot(agg, wihT_ref[...],
                 preferred_element_type=jnp.float32) + bih_ref[...]
    gh = jnp.dot(h, whhT_ref[...],
                 preferred_element_type=jnp.float32) + bhh_ref[...]
    r = jax.nn.sigmoid(gi[:, :D] + gh[:, :D])
    z = jax.nn.sigmoid(gi[:, D:2 * D] + gh[:, D:2 * D])
    n = jnp.tanh(gi[:, 2 * D:] + r * gh[:, 2 * D:])
    x_new = (1.0 - z) * n + z * h
    x_ref[...] = x_new
    m = jnp.dot(x_new, w2_ref[...], preferred_element_type=jnp.float32)
    m_ref[0] = m[:, :DH]
    m_ref[1] = m[:, DH:]


def _gru_mm(agg, h, wihT, whhT, bih, bhh, w2):
    return pl.pallas_call(
        _gru_mm_body,
        grid=(N // BLK,),
        in_specs=[pl.BlockSpec((2, BLK, DH), lambda i: (0, i, 0)),
                  pl.BlockSpec((BLK, D), lambda i: (i, 0)),
                  pl.BlockSpec((D, 3 * D), lambda i: (0, 0)),
                  pl.BlockSpec((D, 3 * D), lambda i: (0, 0)),
                  pl.BlockSpec((1, 3 * D), lambda i: (0, 0)),
                  pl.BlockSpec((1, 3 * D), lambda i: (0, 0)),
                  pl.BlockSpec((D, D), lambda i: (0, 0))],
        out_specs=[pl.BlockSpec((BLK, D), lambda i: (i, 0)),
                   pl.BlockSpec((2, BLK, DH), lambda i: (0, i, 0))],
        out_shape=[jax.ShapeDtypeStruct((N, D), jnp.float32),
                   jax.ShapeDtypeStruct((2, N, DH), jnp.float32)],
    )(agg, h, wihT, whhT, bih, bhh, w2)


def _gru_sum_body(agg_ref, h_ref, wihT_ref, whhT_ref, bih_ref, bhh_ref,
                  o_ref):
    agg = jnp.concatenate([agg_ref[0], agg_ref[1]], axis=1)
    h = h_ref[...]
    gi = jnp.dot(agg, wihT_ref[...],
                 preferred_element_type=jnp.float32) + bih_ref[...]
    gh = jnp.dot(h, whhT_ref[...],
                 preferred_element_type=jnp.float32) + bhh_ref[...]
    r = jax.nn.sigmoid(gi[:, :D] + gh[:, :D])
    z = jax.nn.sigmoid(gi[:, D:2 * D] + gh[:, D:2 * D])
    n = jnp.tanh(gi[:, 2 * D:] + r * gh[:, 2 * D:])
    x_new = (1.0 - z) * n + z * h

    @pl.when(pl.program_id(0) == 0)
    def _init():
        o_ref[...] = jnp.zeros_like(o_ref)

    o_ref[...] += jnp.sum(x_new, axis=0, keepdims=True)


def _gru_sum(agg, h, wihT, whhT, bih, bhh):
    return pl.pallas_call(
        _gru_sum_body,
        grid=(N // BLK,),
        in_specs=[pl.BlockSpec((2, BLK, DH), lambda i: (0, i, 0)),
                  pl.BlockSpec((BLK, D), lambda i: (i, 0)),
                  pl.BlockSpec((D, 3 * D), lambda i: (0, 0)),
                  pl.BlockSpec((D, 3 * D), lambda i: (0, 0)),
                  pl.BlockSpec((1, 3 * D), lambda i: (0, 0)),
                  pl.BlockSpec((1, 3 * D), lambda i: (0, 0))],
        out_specs=pl.BlockSpec((1, D), lambda i: (0, 0)),
        out_shape=jax.ShapeDtypeStruct((1, D), jnp.float32),
    )(agg, h, wihT, whhT, bih, bhh)


@functools.partial(
    pl.kernel,
    mesh=plsc.VectorSubcoreMesh(core_axis_name="c", subcore_axis_name="s"),
    out_type=jax.ShapeDtypeStruct((2, N, DH), jnp.float32),
    scratch_types=[
        pltpu.VMEM((G, CH), jnp.int32),
        pltpu.VMEM((G, CH), jnp.int32),
        pltpu.VMEM((G, CH), jnp.int32),
        pltpu.VMEM((G, CH), jnp.int32),
        pltpu.VMEM((CH, DH), jnp.float32),
        pltpu.VMEM((CH, DH), jnp.float32),
        pltpu.VMEM((CH, DH), jnp.float32),
        pltpu.VMEM_SHARED((SH_ROWS, DH), jnp.float32),
        pltpu.SemaphoreType.DMA,
        pltpu.SemaphoreType.DMA,
        pltpu.SemaphoreType.DMA,
        pltpu.SemaphoreType.DMA,
        pltpu.SemaphoreType.DMA,
    ],
    compiler_params=pltpu.CompilerParams(use_tc_tiling_on_sc=False),
)
def _sc_scatter(m_hbm, src_hbm, dst_hbm, z_hbm, agg_hbm,
                srcA, dstA, srcB, dstB, rows0, rows1, zbuf,
                shared, sem0, sem1, semA, semB, semZ):
    sc = lax.axis_index("c")
    tid = lax.axis_index("s")
    mview = m_hbm.at[sc]
    aggview = agg_hbm.at[sc]
    rows = (rows0, rows1)
    sems = (sem0, sem1)
    isems = (semA, semB)

    def idx_slices(g):
        base = tid * NGROUP + g
        return (src_hbm.at[pl.ds(base * G, G), :],
                dst_hbm.at[pl.ds(base * G, G), :])

    def load_idx(g, src_b, dst_b, sem):
        s, d = idx_slices(g)
        pltpu.async_copy(s, src_b, sem)
        pltpu.async_copy(d, dst_b, sem)

    def wait_idx(g, src_b, dst_b, sem):
        s, d = idx_slices(g)
        pltpu.make_async_copy(s, src_b, sem).wait()
        pltpu.make_async_copy(d, dst_b, sem).wait()

    load_idx(0, srcA, dstA, semA)
    load_idx(1, srcB, dstB, semB)

    pltpu.async_copy(z_hbm, zbuf, semZ)
    pltpu.make_async_copy(z_hbm, zbuf, semZ).wait()
    for ci in range(25):
        zc = ci * 16 + tid

        @pl.when(zc < SH_ROWS // CH)
        def _():
            pltpu.async_copy(zbuf, shared.at[pl.ds(zc * CH, CH)], semZ)
    for ci in range(25):
        zc = ci * 16 + tid

        @pl.when(zc < SH_ROWS // CH)
        def _():
            pltpu.make_async_copy(zbuf, shared.at[pl.ds(zc * CH, CH)],
                                  semZ).wait()
    plsc.subcore_barrier()

    def fire(src_b, j, p):
        pltpu.async_copy(mview.at[src_b.at[j]], rows[p], sems[p])

    def drain_scatter(src_b, dst_b, j, p):
        pltpu.make_async_copy(mview.at[src_b.at[j]], rows[p],
                              sems[p]).wait()
        pltpu.sync_copy(rows[p], shared.at[dst_b.at[j]], add=True)

    wait_idx(0, srcA, dstA, semA)
    fire(srcA, 0, 0)

    def process(g_next, cs, cd, csem, ns, nd, nsem):
        for j in range(G):
            if j < G - 1:
                fire(cs, j + 1, (j + 1) % 2)
            else:
                wait_idx(g_next, ns, nd, nsem)
                fire(ns, 0, (j + 1) % 2)
            drain_scatter(cs, cd, j, j % 2)

    def body(k, carry):
        process(2 * k + 1, srcA, dstA, semA, srcB, dstB, semB)
        load_idx(2 * k + 2, srcA, dstA, semA)

        @pl.when(k < NGROUP // 2 - 1)
        def _():
            process(2 * k + 2, srcB, dstB, semB, srcA, dstA, semA)
            load_idx(2 * k + 3, srcB, dstB, semB)

        return carry

    lax.fori_loop(0, NGROUP // 2, body, 0)

    process(48, srcB, dstB, semB, srcA, dstA, semA)
    for j in range(G):
        if j < G - 1:
            fire(srcA, j + 1, (j + 1) % 2)
        drain_scatter(srcA, dstA, j, j % 2)
    plsc.subcore_barrier()

    for ci in range(16):
        c = ci * 16 + tid

        @pl.when(c < N // WB)
        def _():
            pltpu.async_copy(shared.at[pl.ds(c * WB, WB)],
                             aggview.at[pl.ds(c * WB, WB)], semZ)
    for ci in range(16):
        c = ci * 16 + tid

        @pl.when(c < N // WB)
        def _():
            pltpu.make_async_copy(shared.at[pl.ds(c * WB, WB)],
                                  aggview.at[pl.ds(c * WB, WB)],
                                  semZ).wait()


@jax.jit
def kernel(h1, edge_index1, weight, w_ih, w_hh, b_ih, b_hh, w_out, b_out):
    src = edge_index1[0]
    dst = edge_index1[1]
    pad = EPAD - E
    pad_src = (jnp.arange(pad, dtype=jnp.int32) * 641) % N
    pad_dst = N + (jnp.arange(pad, dtype=jnp.int32) % NDUMP)
    src_p = jnp.concatenate([src, pad_src]).reshape(EROWS, CH)
    dst_p = jnp.concatenate([dst, pad_dst]).reshape(EROWS, CH)
    zblk = jnp.zeros((CH, DH), jnp.float32)

    wihT = w_ih.T
    whhT = w_hh.T
    bih = b_ih.reshape(1, 3 * D)
    bhh = b_hh.reshape(1, 3 * D)

    m0 = _matmul(h1, weight[0])
    agg0 = _sc_scatter(m0, src_p, dst_p, zblk)
    x1, m1 = _gru_mm(agg0, h1, wihT, whhT, bih, bhh, weight[1])
    agg1 = _sc_scatter(m1, src_p, dst_p, zblk)
    colsum = _gru_sum(agg1, x1, wihT, whhT, bih, bhh)

    a2 = colsum @ w_out.T + N * b_out
    return jax.nn.softmax(a2, axis=-1)


# speedup vs baseline: 8.9615x; 1.0317x over previous
import functools
import jax
import jax.numpy as jnp
from jax import lax
from jax.experimental import pallas as pl
from jax.experimental.pallas import tpu as pltpu
from jax.experimental.pallas import tpu_sc as plsc

N = 50000
E = 800000
D = 64
DH = D // 2

BLK = 5000

CH = 128
NSUB = 16
G = 8
NGROUP = 49
NCHUNK = G * NGROUP
EPAD = CH * NSUB * NCHUNK
EROWS = EPAD // CH
SH_ROWS = 50176
NDUMP = SH_ROWS - N
WB = 200


def _mm_body(x_ref, w_ref, o_ref):
    r = jnp.dot(x_ref[...], w_ref[...],
                preferred_element_type=jnp.float32)
    o_ref[0] = r[:, :DH]
    o_ref[1] = r[:, DH:]


def _matmul(x, w):
    return pl.pallas_call(
        _mm_body,
        grid=(N // BLK,),
        in_specs=[pl.BlockSpec((BLK, D), lambda i: (i, 0)),
                  pl.BlockSpec((D, D), lambda i: (0, 0))],
        out_specs=pl.BlockSpec((2, BLK, DH), lambda i: (0, i, 0)),
        out_shape=jax.ShapeDtypeStruct((2, N, DH), jnp.float32),
    )(x, w)


def _gru_mm_body(agg_ref, h_ref, wihT_ref, whhT_ref, bih_ref, bhh_ref,
                 w2_ref, x_ref, m_ref):
    agg = jnp.concatenate([agg_ref[0], agg_ref[1]], axis=1)
    h = h_ref[...]
    gi = jnp.dot(agg, wihT_ref[...],
                 preferred_element_type=jnp.float32) + bih_ref[...]
    gh = jnp.dot(h, whhT_ref[...],
                 preferred_element_type=jnp.float32) + bhh_ref[...]
    r = jax.nn.sigmoid(gi[:, :D] + gh[:, :D])
    z = jax.nn.sigmoid(gi[:, D:2 * D] + gh[:, D:2 * D])
    n = jnp.tanh(gi[:, 2 * D:] + r * gh[:, 2 * D:])
    x_new = (1.0 - z) * n + z * h
    x_ref[...] = x_new
    m = jnp.dot(x_new, w2_ref[...], preferred_element_type=jnp.float32)
    m_ref[0] = m[:, :DH]
    m_ref[1] = m[:, DH:]


def _gru_mm(agg, h, wihT, whhT, bih, bhh, w2):
    return pl.pallas_call(
        _gru_mm_body,
        grid=(N // BLK,),
        in_specs=[pl.BlockSpec((2, BLK, DH), lambda i: (0, i, 0)),
                  pl.BlockSpec((BLK, D), lambda i: (i, 0)),
                  pl.BlockSpec((D, 3 * D), lambda i: (0, 0)),
                  pl.BlockSpec((D, 3 * D), lambda i: (0, 0)),
                  pl.BlockSpec((1, 3 * D), lambda i: (0, 0)),
                  pl.BlockSpec((1, 3 * D), lambda i: (0, 0)),
                  pl.BlockSpec((D, D), lambda i: (0, 0))],
        out_specs=[pl.BlockSpec((BLK, D), lambda i: (i, 0)),
                   pl.BlockSpec((2, BLK, DH), lambda i: (0, i, 0))],
        out_shape=[jax.ShapeDtypeStruct((N, D), jnp.float32),
                   jax.ShapeDtypeStruct((2, N, DH), jnp.float32)],
    )(agg, h, wihT, whhT, bih, bhh, w2)


def _gru_sum_body(agg_ref, h_ref, wihT_ref, whhT_ref, bih_ref, bhh_ref,
                  o_ref):
    agg = jnp.concatenate([agg_ref[0], agg_ref[1]], axis=1)
    h = h_ref[...]
    gi = jnp.dot(agg, wihT_ref[...],
                 preferred_element_type=jnp.float32) + bih_ref[...]
    gh = jnp.dot(h, whhT_ref[...],
                 preferred_element_type=jnp.float32) + bhh_ref[...]
    r = jax.nn.sigmoid(gi[:, :D] + gh[:, :D])
    z = jax.nn.sigmoid(gi[:, D:2 * D] + gh[:, D:2 * D])
    n = jnp.tanh(gi[:, 2 * D:] + r * gh[:, 2 * D:])
    x_new = (1.0 - z) * n + z * h

    @pl.when(pl.program_id(0) == 0)
    def _init():
        o_ref[...] = jnp.zeros_like(o_ref)

    o_ref[...] += jnp.sum(x_new, axis=0, keepdims=True)


def _gru_sum(agg, h, wihT, whhT, bih, bhh):
    return pl.pallas_call(
        _gru_sum_body,
        grid=(N // BLK,),
        in_specs=[pl.BlockSpec((2, BLK, DH), lambda i: (0, i, 0)),
                  pl.BlockSpec((BLK, D), lambda i: (i, 0)),
                  pl.BlockSpec((D, 3 * D), lambda i: (0, 0)),
                  pl.BlockSpec((D, 3 * D), lambda i: (0, 0)),
                  pl.BlockSpec((1, 3 * D), lambda i: (0, 0)),
                  pl.BlockSpec((1, 3 * D), lambda i: (0, 0))],
        out_specs=pl.BlockSpec((1, D), lambda i: (0, 0)),
        out_shape=jax.ShapeDtypeStruct((1, D), jnp.float32),
    )(agg, h, wihT, whhT, bih, bhh)


@functools.partial(
    pl.kernel,
    mesh=plsc.VectorSubcoreMesh(core_axis_name="c", subcore_axis_name="s"),
    out_type=jax.ShapeDtypeStruct((2, N, DH), jnp.float32),
    scratch_types=[
        pltpu.VMEM((G, CH), jnp.int32),
        pltpu.VMEM((G, CH), jnp.int32),
        pltpu.VMEM((G, CH), jnp.int32),
        pltpu.VMEM((G, CH), jnp.int32),
        pltpu.VMEM((CH, DH), jnp.float32),
        pltpu.VMEM((CH, DH), jnp.float32),
        pltpu.VMEM((CH, DH), jnp.float32),
        pltpu.VMEM_SHARED((SH_ROWS, DH), jnp.float32),
        pltpu.SemaphoreType.DMA,
        pltpu.SemaphoreType.DMA,
        pltpu.SemaphoreType.DMA,
        pltpu.SemaphoreType.DMA,
        pltpu.SemaphoreType.DMA,
    ],
    compiler_params=pltpu.CompilerParams(use_tc_tiling_on_sc=False),
)
def _sc_scatter(m_hbm, src_hbm, dst_hbm, z_hbm, agg_hbm,
                srcA, dstA, srcB, dstB, rows0, rows1, zbuf,
                shared, sem0, sem1, semA, semB, semZ):
    sc = lax.axis_index("c")
    tid = lax.axis_index("s")
    mview = m_hbm.at[sc]
    aggview = agg_hbm.at[sc]
    rows = (rows0, rows1)
    sems = (sem0, sem1)
    isems = (semA, semB)

    def idx_slices(g):
        base = tid * NGROUP + g
        return (src_hbm.at[pl.ds(base * G, G), :],
                dst_hbm.at[pl.ds(base * G, G), :])

    def load_idx(g, src_b, dst_b, sem):
        s, d = idx_slices(g)
        pltpu.async_copy(s, src_b, sem)
        pltpu.async_copy(d, dst_b, sem)

    def wait_idx(g, src_b, dst_b, sem):
        s, d = idx_slices(g)
        pltpu.make_async_copy(s, src_b, sem).wait()
        pltpu.make_async_copy(d, dst_b, sem).wait()

    load_idx(0, srcA, dstA, semA)
    load_idx(1, srcB, dstB, semB)

    pltpu.async_copy(z_hbm, zbuf, semZ)
    pltpu.make_async_copy(z_hbm, zbuf, semZ).wait()
    for ci in range(25):
        zc = ci * 16 + tid

        @pl.when(zc < SH_ROWS // CH)
        def _():
            pltpu.async_copy(zbuf, shared.at[pl.ds(zc * CH, CH)], semZ)
    for ci in range(25):
        zc = ci * 16 + tid

        @pl.when(zc < SH_ROWS // CH)
        def _():
            pltpu.make_async_copy(zbuf, shared.at[pl.ds(zc * CH, CH)],
                                  semZ).wait()
    plsc.subcore_barrier()

    def fire(src_b, j, p):
        pltpu.async_copy(mview.at[src_b.at[j]], rows[p], sems[p])

    def drain_scatter(src_b, dst_b, j, p):
        pltpu.make_async_copy(mview.at[src_b.at[j]], rows[p],
                              sems[p]).wait()
        pltpu.sync_copy(rows[p], shared.at[dst_b.at[j]], add=True)

    wait_idx(0, srcA, dstA, semA)
    fire(srcA, 0, 0)

    def process(g_next, cs, cd, csem, ns, nd, nsem):
        for j in range(G):
            if j < G - 1:
                fire(cs, j + 1, (j + 1) % 2)
            else:
                wait_idx(g_next, ns, nd, nsem)
                fire(ns, 0, (j + 1) % 2)
            drain_scatter(cs, cd, j, j % 2)

    def body(k, carry):
        process(2 * k + 1, srcA, dstA, semA, srcB, dstB, semB)
        load_idx(2 * k + 2, srcA, dstA, semA)

        @pl.when(k < NGROUP // 2 - 1)
        def _():
            process(2 * k + 2, srcB, dstB, semB, srcA, dstA, semA)
            load_idx(2 * k + 3, srcB, dstB, semB)

        return carry

    lax.fori_loop(0, NGROUP // 2, body, 0)

    process(48, srcB, dstB, semB, srcA, dstA, semA)
    for j in range(G):
        if j < G - 1:
            fire(srcA, j + 1, (j + 1) % 2)
        drain_scatter(srcA, dstA, j, j % 2)
    plsc.subcore_barrier()

    for ci in range(16):
        c = ci * 16 + tid

        @pl.when(c < N // WB)
        def _():
            pltpu.async_copy(shared.at[pl.ds(c * WB, WB)],
                             aggview.at[pl.ds(c * WB, WB)], semZ)
    for ci in range(16):
        c = ci * 16 + tid

        @pl.when(c < N // WB)
        def _():
            pltpu.make_async_copy(shared.at[pl.ds(c * WB, WB)],
                                  aggview.at[pl.ds(c * WB, WB)],
                                  semZ).wait()


@jax.jit
def kernel(h1, edge_index1, weight, w_ih, w_hh, b_ih, b_hh, w_out, b_out):
    src = edge_index1[0]
    dst = edge_index1[1]
    pad = EPAD - E
    pad_src = (jnp.arange(pad, dtype=jnp.int32) * 641) % N
    pad_dst = N + (jnp.arange(pad, dtype=jnp.int32) % NDUMP)
    src_p = jnp.concatenate([src, pad_src]).reshape(EROWS, CH)
    dst_p = jnp.concatenate([dst, pad_dst]).reshape(EROWS, CH)
    zblk = jnp.zeros((CH, DH), jnp.float32)

    wihT = w_ih.T
    whhT = w_hh.T
    bih = b_ih.reshape(1, 3 * D)
    bhh = b_hh.reshape(1, 3 * D)

    m0 = _matmul(h1, weight[0])
    agg0 = _sc_scatter(m0, src_p, dst_p, zblk)
    x1, m1 = _gru_mm(agg0, h1, wihT, whhT, bih, bhh, weight[1])
    agg1 = _sc_scatter(m1, src_p, dst_p, zblk)
    colsum = _gru_sum(agg1, x1, wihT, whhT, bih, bhh)

    a2 = colsum @ w_out.T + N * b_out
    return jax.nn.softmax(a2, axis=-1)
